# baseline (speedup 1.0000x reference)
import functools

import jax
import jax.numpy as jnp
from jax import lax
from jax.experimental import pallas as pl
from jax.experimental.pallas import tpu as pltpu
from jax.experimental.pallas import tpu_sc as plsc


def _build(B, D, NTOP):
    info = plsc.get_sparse_core_info()
    NC, NS, L = info.num_cores, info.num_subcores, info.num_lanes
    NW = NC * NS
    BPW = B // NW
    CB = 64
    NCH = BPW // CB
    ROWS = CB * NTOP
    NG = ROWS // 128
    NL = D // L

    mesh = plsc.VectorSubcoreMesh(core_axis_name="c", subcore_axis_name="s")

    @functools.partial(
        pl.kernel,
        mesh=mesh,
        out_type=jax.ShapeDtypeStruct((B, D), jnp.float32),
        compiler_params=pltpu.CompilerParams(
            use_tc_tiling_on_sc=False, needs_layout_passes=False),
        scratch_types=[
            pltpu.VMEM((CB,), jnp.int32),
            pltpu.VMEM((ROWS,), jnp.int32),
            pltpu.VMEM((NG, 128), jnp.int32),
            pltpu.VMEM((ROWS, D), jnp.float32),
            pltpu.VMEM((CB, D), jnp.float32),
            pltpu.VMEM((CB, D), jnp.float32),
            pltpu.SemaphoreType.DMA,
        ],
    )
    def sc_kernel(u_hbm, ue_hbm, ie_hbm, idx_hbm, out_hbm,
                  u_c, idx_c, pidx, items_v, ue_c, out_c, sem):
        wid = lax.axis_index("s") * NC + lax.axis_index("c")
        base = wid * BPW

        def chunk_body(cb, carry):
            off = pl.multiple_of(cb * CB, CB)
            pltpu.sync_copy(u_hbm.at[pl.ds(base + off, CB)], u_c)
            pltpu.sync_copy(
                idx_hbm.at[pl.ds((base + off) * NTOP, ROWS)], idx_c)
            ue_cp = pltpu.async_copy(ue_hbm.at[u_c], ue_c, sem)

            for k in range(ROWS // L):
                pidx[k // 8, pl.ds((k % 8) * L, L)] = idx_c[pl.ds(k * L, L)]

            cps = []
            for g in range(NG):
                cps.append(pltpu.async_copy(
                    ie_hbm.at[pidx.at[g]],
                    items_v.at[pl.ds(g * 128, 128), :], sem))
            ue_cp.wait()

            def bbody(b, carry2):
                rb = b * NTOP
                acc = [jnp.zeros((L,), jnp.float32) for _ in range(NL)]
                cnt = jnp.float32(0.0)
                for j in range(NTOP):
                    r = [items_v[rb + j, pl.ds(c * L, L)] for c in range(NL)]
                    for c in range(NL):
                        acc[c] = acc[c] + r[c]
                    s = (r[0] + r[1]) + (r[2] + r[3])
                    rs = jnp.sum(s)
                    cnt = cnt + (rs != 0.0).astype(jnp.float32)
                dv = lax.broadcast_in_dim(cnt + 1e-12, (L,), ())
                for c in range(NL):
                    out_c[b, pl.ds(c * L, L)] = (
                        ue_c[b, pl.ds(c * L, L)] + acc[c] / dv)
                return carry2

            bend = 0
            for g in range(NG):
                cps[g].wait()
                bstart, bend = bend, (128 * (g + 1)) // NTOP
                lax.fori_loop(bstart, min(bend, CB), bbody, 0)
            pltpu.sync_copy(out_c, out_hbm.at[pl.ds(base + off, CB)])
            return carry

        lax.fori_loop(0, NCH, chunk_body, 0)

    return sc_kernel


@functools.lru_cache(maxsize=None)
def _built(B, D, NTOP):
    return _build(B, D, NTOP)


def kernel(u, user_emb, item_emb, user_top_index):
    B = u.shape[0]
    D = user_emb.shape[1]
    NTOP = user_top_index.shape[1]
    u = u.astype(jnp.int32)
    uti_flat = user_top_index.astype(jnp.int32).reshape(-1)
    pos = (u[:, None] * NTOP + jnp.arange(NTOP, dtype=jnp.int32)).reshape(-1)
    idx = uti_flat.at[pos].get(mode="promise_in_bounds")
    return _built(B, D, NTOP)(u, user_emb, item_emb, idx)


# speedup vs baseline: 1.0038x; 1.0038x over previous
import functools

import jax
import jax.numpy as jnp
from jax import lax
from jax.experimental import pallas as pl
from jax.experimental.pallas import tpu as pltpu
from jax.experimental.pallas import tpu_sc as plsc


def _build(B, D, NTOP):
    info = plsc.get_sparse_core_info()
    NC, NS, L = info.num_cores, info.num_subcores, info.num_lanes
    NW = NC * NS
    BPW = B // NW
    CB = 64
    NCH = BPW // CB
    ROWS = CB * NTOP
    NG = ROWS // 128
    NL = D // L

    mesh = plsc.VectorSubcoreMesh(core_axis_name="c", subcore_axis_name="s")

    @functools.partial(
        pl.kernel,
        mesh=mesh,
        out_type=jax.ShapeDtypeStruct((B, D), jnp.float32),
        compiler_params=pltpu.CompilerParams(
            use_tc_tiling_on_sc=False, needs_layout_passes=False),
        scratch_types=[
            pltpu.VMEM((CB,), jnp.int32),
            pltpu.VMEM((ROWS,), jnp.int32),
            pltpu.VMEM((NG, 128), jnp.int32),
            pltpu.VMEM((ROWS, D), jnp.float32),
            pltpu.VMEM((CB, D), jnp.float32),
            pltpu.VMEM((CB, D), jnp.float32),
            pltpu.SemaphoreType.DMA,
        ],
    )
    def sc_kernel(u_hbm, ue_hbm, ie_hbm, idx_hbm, out_hbm,
                  u_c, idx_c, pidx, items_v, ue_c, out_c, sem):
        wid = lax.axis_index("s") * NC + lax.axis_index("c")
        base = wid * BPW

        def chunk_body(cb, carry):
            off = pl.multiple_of(cb * CB, CB)
            pltpu.sync_copy(u_hbm.at[pl.ds(base + off, CB)], u_c)
            pltpu.sync_copy(
                idx_hbm.at[pl.ds((base + off) * NTOP, ROWS)], idx_c)
            ue_cp = pltpu.async_copy(ue_hbm.at[u_c], ue_c, sem)

            for k in range(ROWS // L):
                pidx[k // 8, pl.ds((k % 8) * L, L)] = idx_c[pl.ds(k * L, L)]

            cps = []
            for g in range(NG):
                cps.append(pltpu.async_copy(
                    ie_hbm.at[pidx.at[g]],
                    items_v.at[pl.ds(g * 128, 128), :], sem))
            ue_cp.wait()

            def bbody(b, carry2):
                rb = b * NTOP
                acc = [jnp.zeros((L,), jnp.float32) for _ in range(NL)]
                cnt = jnp.float32(0.0)
                for j in range(NTOP):
                    r = [items_v[rb + j, pl.ds(c * L, L)] for c in range(NL)]
                    for c in range(NL):
                        acc[c] = acc[c] + r[c]
                    s = (r[0] + r[1]) + (r[2] + r[3])
                    rs = jnp.sum(s)
                    cnt = cnt + (rs != 0.0).astype(jnp.float32)
                dv = lax.broadcast_in_dim(cnt + 1e-12, (L,), ())
                for c in range(NL):
                    out_c[b, pl.ds(c * L, L)] = (
                        ue_c[b, pl.ds(c * L, L)] + acc[c] / dv)
                return carry2

            bend = 0
            for g in range(NG):
                cps[g].wait()
                bstart, bend = bend, (128 * (g + 1)) // NTOP
                lax.fori_loop(bstart, min(bend, CB), bbody, 0)
            pltpu.sync_copy(out_c, out_hbm.at[pl.ds(base + off, CB)])
            return carry

        lax.fori_loop(0, NCH, chunk_body, 0)

    return sc_kernel


@functools.lru_cache(maxsize=None)
def _built(B, D, NTOP):
    return _build(B, D, NTOP)


def kernel(u, user_emb, item_emb, user_top_index):
    B = u.shape[0]
    D = user_emb.shape[1]
    NTOP = user_top_index.shape[1]
    u = u.astype(jnp.int32)
    idx = jnp.take(user_top_index.astype(jnp.int32), u, axis=0).reshape(-1)
    return _built(B, D, NTOP)(u, user_emb, item_emb, idx)
